# baseline (speedup 1.0000x reference)
import functools

import jax
import jax.numpy as jnp
from jax import lax
from jax.experimental import pallas as pl
from jax.experimental.pallas import tpu as pltpu
from jax.experimental.pallas import tpu_sc as plsc

NUM_USER = 10000
NUM_ITEM = 10000
DIM = 128
HDIM = DIM // 2
NUM_EDGES = 320000
NUM_PAIRS = 4096

NC = 2
NS = 16
LANES = 16
NW = NC * NS
EPT = NUM_EDGES // NS
CHUNK = 80
NCH = EPT // CHUNK
NBUF = 5
CNTW = 16
PPW = NUM_PAIRS // NW
OWN = 640
OWN_LAST = NUM_ITEM - OWN * (NS - 1)
CORE_ROWS = NUM_ITEM // NC
OWNC = 320
OWNC_LAST = CORE_ROWS - OWNC * (NS - 1)
ZB = 128


def _make_agg():
    mesh = plsc.VectorSubcoreMesh(core_axis_name="c", subcore_axis_name="s",
                                  num_cores=NC, num_subcores=NS)

    @functools.partial(
        pl.kernel,
        out_type=[
            jax.ShapeDtypeStruct((NC, NUM_ITEM, HDIM), jnp.float32),
            jax.ShapeDtypeStruct((NUM_ITEM, CNTW), jnp.float32),
            jax.ShapeDtypeStruct((NC, NUM_USER, HDIM), jnp.float32),
            jax.ShapeDtypeStruct((NUM_USER, CNTW), jnp.float32),
        ],
        mesh=mesh,
        scratch_types=[
            pltpu.VMEM((NCH, 1, CHUNK), jnp.int32),
            pltpu.VMEM((NCH, 1, CHUNK), jnp.int32),
            pltpu.VMEM((NBUF, CHUNK, HDIM), jnp.float32),
            pltpu.VMEM((ZB, HDIM), jnp.float32),
            pltpu.VMEM((ZB, CNTW), jnp.float32),
            pltpu.VMEM((CHUNK, CNTW), jnp.float32),
            pltpu.VMEM_SHARED((NUM_ITEM, HDIM), jnp.float32),
            pltpu.VMEM_SHARED((NUM_ITEM, CNTW), jnp.float32),
        ] + [pltpu.SemaphoreType.DMA] * NBUF,
        compiler_params=pltpu.CompilerParams(use_tc_tiling_on_sc=False),
    )
    def agg(edges_u2i, edges_i2u, x_user_h, x_item_h,
            sum_item, cnt_item, sum_user, cnt_user,
            src_v, dst_v, rows_v, zrow_v, zcnt_v, ones_v, acc_sh, cnt_sh,
            *sems):
        c = lax.axis_index("c")
        s = lax.axis_index("s")
        tbase = pl.multiple_of(s * OWN, 8)
        cbase = pl.multiple_of(c * CORE_ROWS + s * OWNC, 8)

        zero16 = jnp.zeros((LANES,), jnp.float32)
        one16 = jnp.ones((LANES,), jnp.float32)

        def zrow_fill(i, carry):
            for k in range(HDIM // LANES):
                zrow_v[i, pl.ds(k * LANES, LANES)] = zero16
            return carry

        lax.fori_loop(0, ZB, zrow_fill, 0)

        def zcnt_fill(i, carry):
            zcnt_v[i, pl.ds(0, LANES)] = zero16
            return carry

        lax.fori_loop(0, ZB, zcnt_fill, 0)

        def ones_fill(i, carry):
            ones_v[i, pl.ds(0, LANES)] = one16
            return carry

        lax.fori_loop(0, CHUNK, ones_fill, 0)

        def copy_blocks(mk_src, mk_dst, nrows):
            r = 0
            while r < nrows:
                n = min(ZB, nrows - r)
                pltpu.sync_copy(mk_src(r, n), mk_dst(r, n))
                r += n

        def split_last(fn, nreg, nlast):
            @pl.when(s < NS - 1)
            def _():
                fn(nreg)

            @pl.when(s == NS - 1)
            def _():
                fn(nlast)

        def run_phase(x_h, edges_hbm, out_sum, out_cnt):
            split_last(lambda nr: copy_blocks(
                lambda r, n: zrow_v.at[pl.ds(0, n)],
                lambda r, n: acc_sh.at[pl.ds(tbase + r, n)], nr), OWN, OWN_LAST)
            split_last(lambda nr: copy_blocks(
                lambda r, n: zcnt_v.at[pl.ds(0, n)],
                lambda r, n: cnt_sh.at[pl.ds(tbase + r, n)], nr), OWN, OWN_LAST)
            plsc.subcore_barrier()

            pltpu.sync_copy(edges_hbm.at[0, s], src_v)
            pltpu.sync_copy(edges_hbm.at[1, s], dst_v)

            def start_gather(j, b):
                @pl.when(c == 0)
                def _():
                    pltpu.async_copy(x_h.at[0].at[src_v.at[j, 0]],
                                     rows_v.at[b], sems[b])

                @pl.when(c == 1)
                def _():
                    pltpu.async_copy(x_h.at[1].at[src_v.at[j, 0]],
                                     rows_v.at[b], sems[b])

            for b in range(NBUF):
                start_gather(b, b)

            def ring_body(i, carry):
                j0 = i * NBUF
                for b in range(NBUF):
                    j = j0 + b
                    pltpu.make_async_copy(x_h.at[0].at[src_v.at[0, 0]],
                                          rows_v.at[b], sems[b]).wait()
                    pltpu.sync_copy(rows_v.at[b], acc_sh.at[dst_v.at[j, 0]],
                                    add=True)
                    pltpu.sync_copy(ones_v, cnt_sh.at[dst_v.at[j, 0]],
                                    add=True)

                    @pl.when(j + NBUF < NCH)
                    def _():
                        start_gather(j + NBUF, b)
                return carry

            lax.fori_loop(0, NCH // NBUF, ring_body, 0)
            plsc.subcore_barrier()

            split_last(lambda nr: copy_blocks(
                lambda r, n: acc_sh.at[pl.ds(tbase + r, n)],
                lambda r, n: out_sum.at[c, pl.ds(tbase + r, n)], nr), OWN, OWN_LAST)
            split_last(lambda nr: copy_blocks(
                lambda r, n: cnt_sh.at[pl.ds(cbase + r, n)],
                lambda r, n: out_cnt.at[pl.ds(cbase + r, n)], nr), OWNC, OWNC_LAST)
            plsc.subcore_barrier()

        run_phase(x_user_h, edges_u2i, sum_item, cnt_item)
        run_phase(x_item_h, edges_i2u, sum_user, cnt_user)

    return agg


RBLK = 2000


def _dense_block(x_ref, s_ref, c_ref, w_self_ref, w_agg_ref, o_ref):
    ssum = jnp.concatenate([s_ref[0], s_ref[1]], axis=-1)
    cnt = c_ref[:, 0:1]
    agg = ssum / jnp.clip(cnt, 1.0, None)
    z = (lax.dot_general(x_ref[...], w_self_ref[...], (((1,), (0,)), ((), ())),
                         precision=lax.Precision.HIGHEST)
         + lax.dot_general(agg, w_agg_ref[...], (((1,), (0,)), ((), ())),
                           precision=lax.Precision.HIGHEST))
    o_ref[...] = jnp.maximum(z, 0.0)


def _dense_body(xu, su, cu, wus, wiu, xi, si, ci, wis, wui, zu, zi):
    _dense_block(xu, su, cu, wus, wiu, zu)
    _dense_block(xi, si, ci, wis, wui, zi)


def _make_dense():
    row_spec = pl.BlockSpec((RBLK, DIM), lambda i: (i, 0))
    sum_spec = pl.BlockSpec((NC, RBLK, HDIM), lambda i: (0, i, 0))
    cnt_spec = pl.BlockSpec((RBLK, CNTW), lambda i: (i, 0))
    w_spec = pl.BlockSpec((DIM, DIM), lambda i: (0, 0))
    return pl.pallas_call(
        _dense_body,
        grid=(NUM_USER // RBLK,),
        in_specs=[row_spec, sum_spec, cnt_spec, w_spec, w_spec,
                  row_spec, sum_spec, cnt_spec, w_spec, w_spec],
        out_specs=[row_spec, row_spec],
        out_shape=[jax.ShapeDtypeStruct((NUM_USER, DIM), jnp.float32),
                   jax.ShapeDtypeStruct((NUM_ITEM, DIM), jnp.float32)],
    )


def _make_score():
    mesh = plsc.VectorSubcoreMesh(core_axis_name="c", subcore_axis_name="s",
                                  num_cores=NC, num_subcores=NS)

    @functools.partial(
        pl.kernel,
        out_type=jax.ShapeDtypeStruct((NUM_PAIRS,), jnp.float32),
        mesh=mesh,
        scratch_types=[
            pltpu.VMEM((PPW,), jnp.int32),
            pltpu.VMEM((PPW,), jnp.int32),
            pltpu.VMEM((PPW, DIM), jnp.float32),
            pltpu.VMEM((PPW, DIM), jnp.float32),
            pltpu.VMEM((PPW * LANES,), jnp.float32),
            pltpu.VMEM((PPW,), jnp.float32),
            pltpu.SemaphoreType.DMA,
            pltpu.SemaphoreType.DMA,
        ],
        compiler_params=pltpu.CompilerParams(needs_layout_passes=False),
    )
    def score(row_hbm, col_hbm, zu_hbm, zi_hbm, out_hbm,
              row_v, col_v, zu_v, zi_v, part_v, out_v, sem_a, sem_b):
        c = lax.axis_index("c")
        s = lax.axis_index("s")
        w = s * NC + c
        base = pl.multiple_of(w * PPW, 8)
        pltpu.sync_copy(row_hbm.at[pl.ds(base, PPW)], row_v)
        pltpu.sync_copy(col_hbm.at[pl.ds(base, PPW)], col_v)
        cp_a = pltpu.async_copy(zu_hbm.at[row_v], zu_v, sem_a)
        cp_b = pltpu.async_copy(zi_hbm.at[col_v], zi_v, sem_b)
        cp_a.wait()
        cp_b.wait()

        def pair_body(p, carry):
            acc = zu_v[p, pl.ds(0, LANES)] * zi_v[p, pl.ds(0, LANES)]
            for k in range(1, DIM // LANES):
                acc = acc + zu_v[p, pl.ds(k * LANES, LANES)] * zi_v[p, pl.ds(k * LANES, LANES)]
            part_v[pl.ds(p * LANES, LANES)] = acc
            return carry

        lax.fori_loop(0, PPW, pair_body, 0)

        for g in range(PPW // LANES):
            ids = lax.iota(jnp.int32, LANES) + g * LANES
            flat = ids * LANES
            tot = plsc.load_gather(part_v, [flat])
            for k in range(1, LANES):
                tot = tot + plsc.load_gather(part_v, [flat + k])
            out_v[pl.ds(g * LANES, LANES)] = tot
        pltpu.sync_copy(out_v, out_hbm.at[pl.ds(base, PPW)])

    return score


def kernel(x_user, x_item, edge_index_u2i, edge_index_i2u, edge_label_index,
           W_user_self, W_item_self, W_u2i, W_i2u):
    e_u2i = edge_index_u2i.astype(jnp.int32).reshape(2, NS, NCH, 1, CHUNK)
    e_i2u = edge_index_i2u.astype(jnp.int32).reshape(2, NS, NCH, 1, CHUNK)
    eli = edge_label_index.astype(jnp.int32)
    x_user_h = jnp.transpose(x_user.reshape(NUM_USER, 2, HDIM), (1, 0, 2))
    x_item_h = jnp.transpose(x_item.reshape(NUM_ITEM, 2, HDIM), (1, 0, 2))

    agg = _make_agg()
    si, ci, su, cu = agg(e_u2i, e_i2u, x_user_h, x_item_h)

    dense = _make_dense()
    z_user, z_item = dense(x_user, su, cu, W_user_self, W_i2u,
                           x_item, si, ci, W_item_self, W_u2i)

    score = _make_score()
    return score(eli[0], eli[1], z_user, z_item)


# speedup vs baseline: 1.0248x; 1.0248x over previous
import functools

import jax
import jax.numpy as jnp
from jax import lax
from jax.experimental import pallas as pl
from jax.experimental.pallas import tpu as pltpu
from jax.experimental.pallas import tpu_sc as plsc

NUM_USER = 10000
NUM_ITEM = 10000
DIM = 128
HDIM = DIM // 2
NUM_EDGES = 320000
NUM_PAIRS = 4096

NC = 2
NS = 16
LANES = 16
NW = NC * NS
EPT = NUM_EDGES // NS
CHUNK = 80
NCH = EPT // CHUNK
NBUF = 5
CNTW = 16
PPW = NUM_PAIRS // NW
OWN = 640
OWN_LAST = NUM_ITEM - OWN * (NS - 1)
CORE_ROWS = NUM_ITEM // NC
OWNC = 320
OWNC_LAST = CORE_ROWS - OWNC * (NS - 1)
ZB = 128


def _make_agg():
    mesh = plsc.VectorSubcoreMesh(core_axis_name="c", subcore_axis_name="s",
                                  num_cores=NC, num_subcores=NS)

    @functools.partial(
        pl.kernel,
        out_type=[
            jax.ShapeDtypeStruct((NC, NUM_ITEM, HDIM), jnp.float32),
            jax.ShapeDtypeStruct((NUM_ITEM, CNTW), jnp.float32),
            jax.ShapeDtypeStruct((NC, NUM_USER, HDIM), jnp.float32),
            jax.ShapeDtypeStruct((NUM_USER, CNTW), jnp.float32),
        ],
        mesh=mesh,
        scratch_types=[
            pltpu.VMEM((NCH, 1, CHUNK), jnp.int32),
            pltpu.VMEM((NCH, 1, CHUNK), jnp.int32),
            pltpu.VMEM((NBUF, CHUNK, HDIM), jnp.float32),
            pltpu.VMEM((ZB, HDIM), jnp.float32),
            pltpu.VMEM((ZB, CNTW), jnp.float32),
            pltpu.VMEM((CHUNK, CNTW), jnp.float32),
            pltpu.VMEM_SHARED((NUM_ITEM, HDIM), jnp.float32),
            pltpu.VMEM_SHARED((NUM_ITEM, CNTW), jnp.float32),
        ] + [pltpu.SemaphoreType.DMA] * NBUF,
        compiler_params=pltpu.CompilerParams(use_tc_tiling_on_sc=False),
    )
    def agg(edges_u2i, edges_i2u,
            x_user_lo, x_user_hi, x_item_lo, x_item_hi,
            sum_item, cnt_item, sum_user, cnt_user,
            src_v, dst_v, rows_v, zrow_v, zcnt_v, ones_v, acc_sh, cnt_sh,
            *sems):
        c = lax.axis_index("c")
        s = lax.axis_index("s")
        tbase = pl.multiple_of(s * OWN, 8)
        cbase = pl.multiple_of(c * CORE_ROWS + s * OWNC, 8)

        zero16 = jnp.zeros((LANES,), jnp.float32)
        one16 = jnp.ones((LANES,), jnp.float32)

        def zrow_fill(i, carry):
            for k in range(HDIM // LANES):
                zrow_v[i, pl.ds(k * LANES, LANES)] = zero16
            return carry

        lax.fori_loop(0, ZB, zrow_fill, 0)

        def zcnt_fill(i, carry):
            zcnt_v[i, pl.ds(0, LANES)] = zero16
            return carry

        lax.fori_loop(0, ZB, zcnt_fill, 0)

        def ones_fill(i, carry):
            ones_v[i, pl.ds(0, LANES)] = one16
            return carry

        lax.fori_loop(0, CHUNK, ones_fill, 0)

        def copy_blocks(mk_src, mk_dst, nrows):
            r = 0
            while r < nrows:
                n = min(ZB, nrows - r)
                pltpu.sync_copy(mk_src(r, n), mk_dst(r, n))
                r += n

        def split_last(fn, nreg, nlast):
            @pl.when(s < NS - 1)
            def _():
                fn(nreg)

            @pl.when(s == NS - 1)
            def _():
                fn(nlast)

        def run_phase(x_lo, x_hi, edges_hbm, out_sum, out_cnt):
            split_last(lambda nr: copy_blocks(
                lambda r, n: zrow_v.at[pl.ds(0, n)],
                lambda r, n: acc_sh.at[pl.ds(tbase + r, n)], nr), OWN, OWN_LAST)
            split_last(lambda nr: copy_blocks(
                lambda r, n: zcnt_v.at[pl.ds(0, n)],
                lambda r, n: cnt_sh.at[pl.ds(tbase + r, n)], nr), OWN, OWN_LAST)
            plsc.subcore_barrier()

            pltpu.sync_copy(edges_hbm.at[0, s], src_v)
            pltpu.sync_copy(edges_hbm.at[1, s], dst_v)

            def start_gather(j, b):
                @pl.when(c == 0)
                def _():
                    pltpu.async_copy(x_lo.at[src_v.at[j, 0]], rows_v.at[b],
                                     sems[b])

                @pl.when(c == 1)
                def _():
                    pltpu.async_copy(x_hi.at[src_v.at[j, 0]], rows_v.at[b],
                                     sems[b])

            for b in range(NBUF):
                start_gather(b, b)

            def ring_body(i, carry):
                j0 = i * NBUF
                for b in range(NBUF):
                    j = j0 + b
                    pltpu.make_async_copy(x_lo.at[src_v.at[0, 0]],
                                          rows_v.at[b], sems[b]).wait()
                    pltpu.sync_copy(rows_v.at[b], acc_sh.at[dst_v.at[j, 0]],
                                    add=True)
                    pltpu.sync_copy(ones_v, cnt_sh.at[dst_v.at[j, 0]],
                                    add=True)

                    @pl.when(j + NBUF < NCH)
                    def _():
                        start_gather(j + NBUF, b)
                return carry

            lax.fori_loop(0, NCH // NBUF, ring_body, 0)
            plsc.subcore_barrier()

            split_last(lambda nr: copy_blocks(
                lambda r, n: acc_sh.at[pl.ds(tbase + r, n)],
                lambda r, n: out_sum.at[c, pl.ds(tbase + r, n)], nr), OWN, OWN_LAST)
            split_last(lambda nr: copy_blocks(
                lambda r, n: cnt_sh.at[pl.ds(cbase + r, n)],
                lambda r, n: out_cnt.at[pl.ds(cbase + r, n)], nr), OWNC, OWNC_LAST)
            plsc.subcore_barrier()

        run_phase(x_user_lo, x_user_hi, edges_u2i, sum_item, cnt_item)
        run_phase(x_item_lo, x_item_hi, edges_i2u, sum_user, cnt_user)

    return agg


RBLK = 2000


def _dense_block(x_ref, s_ref, c_ref, w_self_ref, w_agg_ref, o_ref):
    ssum = jnp.concatenate([s_ref[0], s_ref[1]], axis=-1)
    cnt = c_ref[:, 0:1]
    agg = ssum / jnp.clip(cnt, 1.0, None)
    z = (lax.dot_general(x_ref[...], w_self_ref[...], (((1,), (0,)), ((), ())),
                         precision=lax.Precision.HIGHEST)
         + lax.dot_general(agg, w_agg_ref[...], (((1,), (0,)), ((), ())),
                           precision=lax.Precision.HIGHEST))
    o_ref[...] = jnp.maximum(z, 0.0)


def _dense_body(xu, su, cu, wus, wiu, xi, si, ci, wis, wui, zu, zi):
    _dense_block(xu, su, cu, wus, wiu, zu)
    _dense_block(xi, si, ci, wis, wui, zi)


def _make_dense():
    row_spec = pl.BlockSpec((RBLK, DIM), lambda i: (i, 0))
    sum_spec = pl.BlockSpec((NC, RBLK, HDIM), lambda i: (0, i, 0))
    cnt_spec = pl.BlockSpec((RBLK, CNTW), lambda i: (i, 0))
    w_spec = pl.BlockSpec((DIM, DIM), lambda i: (0, 0))
    return pl.pallas_call(
        _dense_body,
        grid=(NUM_USER // RBLK,),
        in_specs=[row_spec, sum_spec, cnt_spec, w_spec, w_spec,
                  row_spec, sum_spec, cnt_spec, w_spec, w_spec],
        out_specs=[row_spec, row_spec],
        out_shape=[jax.ShapeDtypeStruct((NUM_USER, DIM), jnp.float32),
                   jax.ShapeDtypeStruct((NUM_ITEM, DIM), jnp.float32)],
    )


def _make_score():
    mesh = plsc.VectorSubcoreMesh(core_axis_name="c", subcore_axis_name="s",
                                  num_cores=NC, num_subcores=NS)

    @functools.partial(
        pl.kernel,
        out_type=jax.ShapeDtypeStruct((NUM_PAIRS,), jnp.float32),
        mesh=mesh,
        scratch_types=[
            pltpu.VMEM((PPW,), jnp.int32),
            pltpu.VMEM((PPW,), jnp.int32),
            pltpu.VMEM((PPW, DIM), jnp.float32),
            pltpu.VMEM((PPW, DIM), jnp.float32),
            pltpu.VMEM((PPW * LANES,), jnp.float32),
            pltpu.VMEM((PPW,), jnp.float32),
            pltpu.SemaphoreType.DMA,
            pltpu.SemaphoreType.DMA,
        ],
        compiler_params=pltpu.CompilerParams(needs_layout_passes=False),
    )
    def score(row_hbm, col_hbm, zu_hbm, zi_hbm, out_hbm,
              row_v, col_v, zu_v, zi_v, part_v, out_v, sem_a, sem_b):
        c = lax.axis_index("c")
        s = lax.axis_index("s")
        w = s * NC + c
        base = pl.multiple_of(w * PPW, 8)
        pltpu.sync_copy(row_hbm.at[pl.ds(base, PPW)], row_v)
        pltpu.sync_copy(col_hbm.at[pl.ds(base, PPW)], col_v)
        cp_a = pltpu.async_copy(zu_hbm.at[row_v], zu_v, sem_a)
        cp_b = pltpu.async_copy(zi_hbm.at[col_v], zi_v, sem_b)
        cp_a.wait()
        cp_b.wait()

        def pair_body(p, carry):
            acc = zu_v[p, pl.ds(0, LANES)] * zi_v[p, pl.ds(0, LANES)]
            for k in range(1, DIM // LANES):
                acc = acc + zu_v[p, pl.ds(k * LANES, LANES)] * zi_v[p, pl.ds(k * LANES, LANES)]
            part_v[pl.ds(p * LANES, LANES)] = acc
            return carry

        lax.fori_loop(0, PPW, pair_body, 0)

        for g in range(PPW // LANES):
            ids = lax.iota(jnp.int32, LANES) + g * LANES
            flat = ids * LANES
            tot = plsc.load_gather(part_v, [flat])
            for k in range(1, LANES):
                tot = tot + plsc.load_gather(part_v, [flat + k])
            out_v[pl.ds(g * LANES, LANES)] = tot
        pltpu.sync_copy(out_v, out_hbm.at[pl.ds(base, PPW)])

    return score


def kernel(x_user, x_item, edge_index_u2i, edge_index_i2u, edge_label_index,
           W_user_self, W_item_self, W_u2i, W_i2u):
    e_u2i = edge_index_u2i.astype(jnp.int32).reshape(2, NS, NCH, 1, CHUNK)
    e_i2u = edge_index_i2u.astype(jnp.int32).reshape(2, NS, NCH, 1, CHUNK)
    eli = edge_label_index.astype(jnp.int32)
    x_user_lo = x_user[:, :HDIM]
    x_user_hi = x_user[:, HDIM:]
    x_item_lo = x_item[:, :HDIM]
    x_item_hi = x_item[:, HDIM:]

    agg = _make_agg()
    si, ci, su, cu = agg(e_u2i, e_i2u,
                         x_user_lo, x_user_hi, x_item_lo, x_item_hi)

    dense = _make_dense()
    z_user, z_item = dense(x_user, su, cu, W_user_self, W_i2u,
                           x_item, si, ci, W_item_self, W_u2i)

    score = _make_score()
    return score(eli[0], eli[1], z_user, z_item)
